# Initial kernel scaffold; baseline (speedup 1.0000x reference)
import jax
import jax.numpy as jnp
from jax.experimental import pallas as pl


def kernel(h_index, r_index, t_index, hidden_states, rel_hidden_states, x, edge_index, score_text_embs, all_index, rel_table, W0, b0, W1, b1, W_lin, b_lin, W_mlp1, b_mlp1, W_mlp2, b_mlp2):
    raise NotImplementedError("write your pallas kernel here")


# speedup vs baseline: 13.3849x; 13.3849x over previous
import functools
import math

import jax
import jax.numpy as jnp
from jax import lax
from jax.experimental import pallas as pl
from jax.experimental.pallas import tpu as pltpu

_N = 10000
_D = 128
_B = 2
_NEG = 4
_CH = 1024
_NP = 10240


def _select_row(arr, b, nb):
    out = arr[0] * 0.0
    for bb in range(nb):
        out = out + jnp.where(b == bb, 1.0, 0.0) * arr[bb]
    return out


def _dense_body(deg_full_ref, deg_ref, c_ref, s_ref, hsh_ref, rel_ref, sth_ref,
                w0_ref, wl_ref, wm1_ref, wm2_ref, b0_ref, blin_ref, bm1_ref,
                bm2_ref, h0_ref, aggsum_ref, aggmax_ref):
    b = pl.program_id(0)
    i = pl.program_id(1)

    mean_ld = jnp.sum(jnp.log(deg_full_ref[...] + 1.0)) / float(_N)

    deg = deg_ref[...]
    scal = jnp.log(deg + 1.0) / mean_ld
    cb = c_ref[0]
    degc = jnp.maximum(deg, 1.0)

    rel = rel_ref[...]
    hsh = hsh_ref[...]
    sth = sth_ref[...]
    gate0 = jax.nn.sigmoid(jnp.sum(sth * rel, axis=1, keepdims=True)
                           / math.sqrt(float(_D)))
    m = gate0 * hsh * rel
    u = jnp.dot(m, w0_ref[...], preferred_element_type=jnp.float32)
    w = jnp.dot(jnp.maximum(m, 0.0), w0_ref[...],
                preferred_element_type=jnp.float32)
    q = jnp.dot(rel, wl_ref[_D:], preferred_element_type=jnp.float32) \
        + blin_ref[...]
    u_b = _select_row(u, b, _B)[None, :]
    w_b = _select_row(w, b, _B)[None, :]
    q_b = _select_row(q, b, _B)[None, :]
    bnd_b = _select_row(hsh * rel, b, _B)[None, :]
    h0_b = jnp.int32(0)
    for bb in range(_B):
        h0_b = h0_b + jnp.where(b == bb, h0_ref[bb], 0)

    a_mean = scal * cb / degc
    a_coef = a_mean + jnp.where((cb > 0) & (cb == deg), scal, 0.0)
    b_coef = jnp.where((cb > 0) & (cb < deg), scal, 0.0)
    hid1 = jnp.maximum(a_coef * u_b + b_coef * w_b + b0_ref[...], 0.0)
    row_ids = lax.broadcasted_iota(jnp.int32, (_CH, 1), 0) + i * _CH
    hid1 = hid1 + jnp.where(row_ids == h0_b, 1.0, 0.0) * bnd_b

    z1 = jnp.maximum(jnp.dot(hid1, wl_ref[:_D],
                             preferred_element_type=jnp.float32) + q_b, 0.0)
    z2 = jnp.maximum(jnp.dot(z1, wm1_ref[...],
                             preferred_element_type=jnp.float32)
                     + bm1_ref[...], 0.0)
    s1 = jnp.dot(z2, wm2_ref[...], preferred_element_type=jnp.float32) \
        + bm2_ref[...]
    gate1 = jax.nn.sigmoid(s1)
    g = gate1 * hid1

    sb = s_ref[0]
    part_sum = jnp.dot(sb, g, preferred_element_type=jnp.float32)
    neg_inf = jnp.float32(-jnp.inf)
    maxes = []
    for k in range(_NEG):
        mk = (sb[k:k + 1, :] > 0.0)
        gm = jnp.where(jnp.transpose(mk), g, neg_inf)
        maxes.append(jnp.max(gm, axis=0, keepdims=True))
    part_max = jnp.concatenate(maxes, axis=0)

    @pl.when(i == 0)
    def _():
        aggsum_ref[0] = part_sum
        aggmax_ref[0] = part_max

    @pl.when(i > 0)
    def _():
        aggsum_ref[0] = aggsum_ref[0] + part_sum
        aggmax_ref[0] = jnp.maximum(aggmax_ref[0], part_max)


def _final_body(aggsum_ref, aggmax_ref, deg_full_ref, hsh_ref, rel_ref,
                w1_ref, wl_ref, wm1_ref, wm2_ref, b1_ref, blin_ref, bm1_ref,
                bm2_ref, h0_ref, t_ref, out_ref):
    mean_ld = jnp.sum(jnp.log(deg_full_ref[...] + 1.0)) / float(_N)

    degt_rows = []
    for r in range(_B * _NEG):
        degt_rows.append(deg_full_ref[pl.ds(t_ref[r], 1), :])
    degt = jnp.concatenate(degt_rows, axis=0)
    scal_t = jnp.log(degt + 1.0) / mean_ld

    aggsum = aggsum_ref[...].reshape(_B * _NEG, _D)
    aggmax = aggmax_ref[...].reshape(_B * _NEG, _D)
    agg2 = (aggsum / jnp.maximum(degt, 1.0)
            + jnp.where(degt > 0, aggmax, 0.0)) * scal_t

    hid2 = jnp.maximum(jnp.dot(agg2, w1_ref[...],
                               preferred_element_type=jnp.float32)
                       + b1_ref[...], 0.0)

    rel = rel_ref[...]
    hsh = hsh_ref[...]
    bnd = hsh * rel
    q = jnp.dot(rel, wl_ref[_D:], preferred_element_type=jnp.float32) \
        + blin_ref[...]
    rows_b = lax.broadcasted_iota(jnp.int32, (_B * _NEG, 1), 0) // _NEG
    bnd_rows = jnp.zeros((_B * _NEG, _D), jnp.float32)
    q_rows = jnp.zeros((_B * _NEG, _D), jnp.float32)
    tmatch = jnp.zeros((_B * _NEG, 1), jnp.float32)
    for bb in range(_B):
        sel = jnp.where(rows_b == bb, 1.0, 0.0)
        bnd_rows = bnd_rows + sel * bnd[bb][None, :]
        q_rows = q_rows + sel * q[bb][None, :]
        for k in range(_NEG):
            r = bb * _NEG + k
            is_h = jnp.where(t_ref[r] == h0_ref[bb], 1.0, 0.0)
            rsel = jnp.where(lax.broadcasted_iota(
                jnp.int32, (_B * _NEG, 1), 0) == r, 1.0, 0.0)
            tmatch = tmatch + rsel * is_h
    hid2 = hid2 + tmatch * bnd_rows

    z1 = jnp.maximum(jnp.dot(hid2, wl_ref[:_D],
                             preferred_element_type=jnp.float32) + q_rows, 0.0)
    z2 = jnp.maximum(jnp.dot(z1, wm1_ref[...],
                             preferred_element_type=jnp.float32)
                     + bm1_ref[...], 0.0)
    s2 = jnp.dot(z2, wm2_ref[...], preferred_element_type=jnp.float32) \
        + bm2_ref[...]
    out_ref[...] = s2.reshape(_B, _NEG)


def kernel(h_index, r_index, t_index, hidden_states, rel_hidden_states, x,
           edge_index, score_text_embs, all_index, rel_table, W0, b0, W1, b1,
           W_lin, b_lin, W_mlp1, b_mlp1, W_mlp2, b_mlp2):
    e0 = edge_index[0].astype(jnp.int32)
    e1 = edge_index[1].astype(jnp.int32)
    h0 = h_index[:, 0].astype(jnp.int32)
    r0 = r_index[:, 0].astype(jnp.int32)
    t = t_index.astype(jnp.int32).reshape(-1)

    deg = jnp.zeros((_N,), jnp.float32).at[e0].add(1.0).at[e1].add(1.0)

    def ccount(hb):
        return (jnp.zeros((_N,), jnp.float32)
                .at[e1].add(jnp.where(e0 == hb, 1.0, 0.0))
                .at[e0].add(jnp.where(e1 == hb, 1.0, 0.0)))
    c = jax.vmap(ccount)(h0)

    def scount(tv):
        return (jnp.zeros((_N,), jnp.float32)
                .at[e0].add(jnp.where(e1 == tv, 1.0, 0.0))
                .at[e1].add(jnp.where(e0 == tv, 1.0, 0.0)))
    s_rows = jax.vmap(scount)(t)

    pad = _NP - _N
    deg_p = jnp.pad(deg, (0, pad)).reshape(_NP, 1)
    c_p = jnp.pad(c, ((0, 0), (0, pad))).reshape(_B, _NP, 1)
    s_p = jnp.pad(s_rows, ((0, 0), (0, pad))).reshape(_B, _NEG, _NP)

    rel = rel_table[r0]
    hsh = hidden_states[h0]
    sth = score_text_embs[h0]
    b0r = b0.reshape(1, _D)
    b1r = b1.reshape(1, _D)
    blinr = b_lin.reshape(1, _D)
    bm1r = b_mlp1.reshape(1, 2 * _D)
    bm2r = b_mlp2.reshape(1, 1)

    nch = _NP // _CH
    full = lambda shape: pl.BlockSpec(shape, lambda b, i: (0,) * len(shape))
    aggsum, aggmax = pl.pallas_call(
        _dense_body,
        grid=(_B, nch),
        in_specs=[
            full((_NP, 1)),
            pl.BlockSpec((_CH, 1), lambda b, i: (i, 0)),
            pl.BlockSpec((1, _CH, 1), lambda b, i: (b, i, 0)),
            pl.BlockSpec((1, _NEG, _CH), lambda b, i: (b, 0, i)),
            full((_B, _D)), full((_B, _D)), full((_B, _D)),
            full((_D, _D)), full((2 * _D, _D)), full((_D, 2 * _D)),
            full((2 * _D, 1)), full((1, _D)), full((1, _D)),
            full((1, 2 * _D)), full((1, 1)),
            pl.BlockSpec(memory_space=pltpu.SMEM),
        ],
        out_specs=[
            pl.BlockSpec((1, _NEG, _D), lambda b, i: (b, 0, 0)),
            pl.BlockSpec((1, _NEG, _D), lambda b, i: (b, 0, 0)),
        ],
        out_shape=[
            jax.ShapeDtypeStruct((_B, _NEG, _D), jnp.float32),
            jax.ShapeDtypeStruct((_B, _NEG, _D), jnp.float32),
        ],
    )(deg_p, deg_p, c_p, s_p, hsh, rel, sth, W0, W_lin, W_mlp1, W_mlp2,
      b0r, blinr, bm1r, bm2r, h0)

    fullf = lambda shape: pl.BlockSpec(shape, lambda: (0,) * len(shape))
    out = pl.pallas_call(
        _final_body,
        in_specs=[
            fullf((_B, _NEG, _D)), fullf((_B, _NEG, _D)), fullf((_NP, 1)),
            fullf((_B, _D)), fullf((_B, _D)),
            fullf((_D, _D)), fullf((2 * _D, _D)), fullf((_D, 2 * _D)),
            fullf((2 * _D, 1)), fullf((1, _D)), fullf((1, _D)),
            fullf((1, 2 * _D)), fullf((1, 1)),
            pl.BlockSpec(memory_space=pltpu.SMEM),
            pl.BlockSpec(memory_space=pltpu.SMEM),
        ],
        out_specs=fullf((_B, _NEG)),
        out_shape=jax.ShapeDtypeStruct((_B, _NEG), jnp.float32),
    )(aggsum, aggmax, deg_p, hsh, rel, W1, W_lin, W_mlp1, W_mlp2,
      b1r, blinr, bm1r, bm2r, h0, t)
    return out


# speedup vs baseline: 157.9007x; 11.7969x over previous
import math

import jax
import jax.numpy as jnp
from jax import lax
from jax.experimental import pallas as pl
from jax.experimental.pallas import tpu as pltpu
from jax.experimental.pallas import tpu_sc as plsc

_N = 10000
_D = 128
_B = 2
_NEG = 4
_E = 160000
_CH = 1024
_NP = 10240
_NSPEC = _B + _B * _NEG
_NHIST = 1 + _NSPEC
_HROWS = (_NHIST * _N) // 16
_NC = 2
_NSUB = 16
_NW = _NC * _NSUB
_EPW = _E // _NW


_DROWS = _N // 16
_CSLEN = _NSPEC * _N + 16


def _hist_body(e0_hbm, e1_hbm, spec_hbm, zdeg_hbm, zcs_hbm, rowids_hbm,
               out_deg_hbm, out_cs_hbm,
               e0_v, e1_v, spec_v, rowids_v, hist_v, stage_v, ones_v,
               shared_deg, shared_cs):
    cid = lax.axis_index("c")
    sid = lax.axis_index("s")
    wid = sid * _NC + cid
    base = wid * _EPW

    @pl.when(sid == 0)
    def _():
        pltpu.sync_copy(zdeg_hbm, shared_deg)
        pltpu.sync_copy(zcs_hbm, shared_cs)

    pltpu.sync_copy(e0_hbm.at[pl.ds(base, _EPW)], e0_v)
    pltpu.sync_copy(e1_hbm.at[pl.ds(base, _EPW)], e1_v)
    pltpu.sync_copy(spec_hbm, spec_v)
    pltpu.sync_copy(rowids_hbm, rowids_v)
    pltpu.sync_copy(zcs_hbm.at[pl.ds(0, _N)], hist_v)
    ones_v[...] = jnp.ones((16,), jnp.float32)
    plsc.subcore_barrier()

    ones = jnp.ones((16,), jnp.float32)
    lane = lax.iota(jnp.int32, 16)
    specs = [spec_v[s] for s in range(_NSPEC)]

    def body(j, carry):
        valid = lane < (_EPW - j * 16)
        a = e0_v[pl.ds(j * 16, 16)]
        b = e1_v[pl.ds(j * 16, 16)]
        plsc.addupdate_scatter(hist_v, [a], ones, mask=valid)
        plsc.addupdate_scatter(hist_v, [b], ones, mask=valid)

        hit = valid & (a == specs[0])
        for s in range(_NSPEC):
            hit = hit | (valid & (a == specs[s])) | (valid & (b == specs[s]))

        @pl.when(jnp.any(hit))
        def _():
            for s in range(_NSPEC):
                m0 = valid & (a == specs[s])

                @pl.when(jnp.any(m0))
                def _():
                    idx = jnp.where(m0, s * _N + b, _NSPEC * _N + lane)
                    pltpu.sync_copy(ones_v, shared_cs.at[idx], add=True)

                m1 = valid & (b == specs[s])

                @pl.when(jnp.any(m1))
                def _():
                    idx = jnp.where(m1, s * _N + a, _NSPEC * _N + lane)
                    pltpu.sync_copy(ones_v, shared_cs.at[idx], add=True)
        return carry

    lax.fori_loop(0, (_EPW + 15) // 16, body, 0)

    def stage(i, carry):
        stage_v[i] = hist_v[pl.ds(i * 16, 16)]
        return carry

    lax.fori_loop(0, _DROWS, stage, 0)
    pltpu.sync_copy(stage_v, shared_deg.at[rowids_v], add=True)
    plsc.subcore_barrier()

    @pl.when(sid == 0)
    def _():
        pltpu.sync_copy(shared_deg, out_deg_hbm.at[cid])
        pltpu.sync_copy(shared_cs, out_cs_hbm.at[cid])


def _sc_histograms(e0, e1, spec):
    zdeg = jnp.zeros((_DROWS, 16), jnp.float32)
    zcs = jnp.zeros((_CSLEN,), jnp.float32)
    rowids = jnp.arange(_DROWS, dtype=jnp.int32)
    mesh = plsc.VectorSubcoreMesh(core_axis_name="c", subcore_axis_name="s",
                                  num_cores=_NC, num_subcores=_NSUB)
    f = pl.kernel(
        _hist_body,
        out_type=(jax.ShapeDtypeStruct((_NC, _DROWS, 16), jnp.float32),
                  jax.ShapeDtypeStruct((_NC, _CSLEN), jnp.float32)),
        mesh=mesh,
        compiler_params=pltpu.CompilerParams(needs_layout_passes=False),
        scratch_types=[
            pltpu.VMEM((_EPW,), jnp.int32),
            pltpu.VMEM((_EPW,), jnp.int32),
            pltpu.VMEM((_NSPEC, 16), jnp.int32),
            pltpu.VMEM((_DROWS,), jnp.int32),
            pltpu.VMEM((_N,), jnp.float32),
            pltpu.VMEM((_DROWS, 16), jnp.float32),
            pltpu.VMEM((16,), jnp.float32),
            pltpu.VMEM_SHARED((_DROWS, 16), jnp.float32),
            pltpu.VMEM_SHARED((_CSLEN,), jnp.float32),
        ],
    )
    return f(e0, e1, spec, zdeg, zcs, rowids)


def _dense_body(hist_ref, hsh_ref, rel_ref, sth_ref, w0_ref, wl_ref, wm1_ref,
                wm2_ref, b0_ref, blin_ref, bm1_ref, bm2_ref, h0_ref,
                aggsum_ref, aggmax_ref):
    b = pl.program_id(0)
    i = pl.program_id(1)

    deg_full = hist_ref[0, :, 0:1] + hist_ref[1, :, 0:1]
    mean_ld = jnp.sum(jnp.log(deg_full + 1.0)) / float(_N)

    hc = hist_ref[0, pl.ds(i * _CH, _CH), :] \
        + hist_ref[1, pl.ds(i * _CH, _CH), :]
    deg = hc[:, 0:1]
    scal = jnp.log(deg + 1.0) / mean_ld
    degc = jnp.maximum(deg, 1.0)

    rel = rel_ref[...]
    hsh = hsh_ref[...]
    sth = sth_ref[...]
    gate0 = jax.nn.sigmoid(jnp.sum(sth * rel, axis=1, keepdims=True)
                           / math.sqrt(float(_D)))
    m = gate0 * hsh * rel
    u = jnp.dot(m, w0_ref[...], preferred_element_type=jnp.float32)
    w = jnp.dot(jnp.maximum(m, 0.0), w0_ref[...],
                preferred_element_type=jnp.float32)
    q = jnp.dot(rel, wl_ref[_D:], preferred_element_type=jnp.float32) \
        + blin_ref[...]

    zerod = jnp.zeros((1, _D), jnp.float32)
    u_b, w_b, q_b, bnd_b = zerod, zerod, zerod, zerod
    cb = jnp.zeros((_CH, 1), jnp.float32)
    sbt = jnp.zeros((_CH, _NEG), jnp.float32)
    h0_b = jnp.int32(0)
    for bb in range(_B):
        selv = jnp.where(b == bb, 1.0, 0.0)
        u_b = u_b + selv * u[bb][None, :]
        w_b = w_b + selv * w[bb][None, :]
        q_b = q_b + selv * q[bb][None, :]
        bnd_b = bnd_b + selv * (hsh[bb] * rel[bb])[None, :]
        cb = cb + selv * hc[:, 1 + bb:2 + bb]
        sbt = sbt + selv * hc[:, 1 + _B + _NEG * bb:1 + _B + _NEG * (bb + 1)]
        h0_b = h0_b + jnp.where(b == bb, h0_ref[bb], 0)

    a_coef = scal * cb / degc + jnp.where((cb > 0) & (cb == deg), scal, 0.0)
    b_coef = jnp.where((cb > 0) & (cb < deg), scal, 0.0)
    hid1 = jnp.maximum(a_coef * u_b + b_coef * w_b + b0_ref[...], 0.0)
    row_ids = lax.broadcasted_iota(jnp.int32, (_CH, 1), 0) + i * _CH
    hid1 = hid1 + jnp.where(row_ids == h0_b, 1.0, 0.0) * bnd_b

    z1 = jnp.maximum(jnp.dot(hid1, wl_ref[:_D],
                             preferred_element_type=jnp.float32) + q_b, 0.0)
    z2 = jnp.maximum(jnp.dot(z1, wm1_ref[...],
                             preferred_element_type=jnp.float32)
                     + bm1_ref[...], 0.0)
    s1 = jnp.dot(z2, wm2_ref[...], preferred_element_type=jnp.float32) \
        + bm2_ref[...]
    gate1 = jax.nn.sigmoid(s1)
    g = gate1 * hid1

    part_sum = lax.dot_general(sbt, g, (((0,), (0,)), ((), ())),
                               preferred_element_type=jnp.float32)
    neg_inf = jnp.float32(-jnp.inf)
    maxes = []
    for k in range(_NEG):
        gm = jnp.where(sbt[:, k:k + 1] > 0.0, g, neg_inf)
        maxes.append(jnp.max(gm, axis=0, keepdims=True))
    part_max = jnp.concatenate(maxes, axis=0)

    @pl.when(i == 0)
    def _():
        aggsum_ref[0] = part_sum
        aggmax_ref[0] = part_max

    @pl.when(i > 0)
    def _():
        aggsum_ref[0] = aggsum_ref[0] + part_sum
        aggmax_ref[0] = jnp.maximum(aggmax_ref[0], part_max)


def _final_body(aggsum_ref, aggmax_ref, hist_ref, hsh_ref, rel_ref,
                w1_ref, wl_ref, wm1_ref, wm2_ref, b1_ref, blin_ref, bm1_ref,
                bm2_ref, h0_ref, t_ref, out_ref):
    deg_full = hist_ref[0, :, 0:1] + hist_ref[1, :, 0:1]
    mean_ld = jnp.sum(jnp.log(deg_full + 1.0)) / float(_N)

    degt_rows = []
    for r in range(_B * _NEG):
        degt_rows.append(hist_ref[0, pl.ds(t_ref[r], 1), 0:1]
                         + hist_ref[1, pl.ds(t_ref[r], 1), 0:1])
    degt = jnp.concatenate(degt_rows, axis=0)
    scal_t = jnp.log(degt + 1.0) / mean_ld

    aggsum = aggsum_ref[...].reshape(_B * _NEG, _D)
    aggmax = aggmax_ref[...].reshape(_B * _NEG, _D)
    agg2 = (aggsum / jnp.maximum(degt, 1.0)
            + jnp.where(degt > 0, aggmax, 0.0)) * scal_t

    hid2 = jnp.maximum(jnp.dot(agg2, w1_ref[...],
                               preferred_element_type=jnp.float32)
                       + b1_ref[...], 0.0)

    rel = rel_ref[...]
    hsh = hsh_ref[...]
    bnd = hsh * rel
    q = jnp.dot(rel, wl_ref[_D:], preferred_element_type=jnp.float32) \
        + blin_ref[...]
    rows_b = lax.broadcasted_iota(jnp.int32, (_B * _NEG, 1), 0) // _NEG
    row_iota = lax.broadcasted_iota(jnp.int32, (_B * _NEG, 1), 0)
    bnd_rows = jnp.zeros((_B * _NEG, _D), jnp.float32)
    q_rows = jnp.zeros((_B * _NEG, _D), jnp.float32)
    tmatch = jnp.zeros((_B * _NEG, 1), jnp.float32)
    for bb in range(_B):
        sel = jnp.where(rows_b == bb, 1.0, 0.0)
        bnd_rows = bnd_rows + sel * bnd[bb][None, :]
        q_rows = q_rows + sel * q[bb][None, :]
        for k in range(_NEG):
            r = bb * _NEG + k
            is_h = jnp.where(t_ref[r] == h0_ref[bb], 1.0, 0.0)
            tmatch = tmatch + jnp.where(row_iota == r, 1.0, 0.0) * is_h
    hid2 = hid2 + tmatch * bnd_rows

    z1 = jnp.maximum(jnp.dot(hid2, wl_ref[:_D],
                             preferred_element_type=jnp.float32) + q_rows, 0.0)
    z2 = jnp.maximum(jnp.dot(z1, wm1_ref[...],
                             preferred_element_type=jnp.float32)
                     + bm1_ref[...], 0.0)
    s2 = jnp.dot(z2, wm2_ref[...], preferred_element_type=jnp.float32) \
        + bm2_ref[...]
    out_ref[...] = s2.reshape(_B, _NEG)


def kernel(h_index, r_index, t_index, hidden_states, rel_hidden_states, x,
           edge_index, score_text_embs, all_index, rel_table, W0, b0, W1, b1,
           W_lin, b_lin, W_mlp1, b_mlp1, W_mlp2, b_mlp2):
    e0 = edge_index[0].astype(jnp.int32)
    e1 = edge_index[1].astype(jnp.int32)
    h0 = h_index[:, 0].astype(jnp.int32)
    r0 = r_index[:, 0].astype(jnp.int32)
    t = t_index.astype(jnp.int32).reshape(-1)

    spec = jnp.tile(jnp.concatenate([h0, t])[:, None], (1, 16))
    deg_part, cs_part = _sc_histograms(e0, e1, spec)
    hist = jnp.concatenate(
        [deg_part.reshape(_NC, 1, _N),
         cs_part[:, :_NSPEC * _N].reshape(_NC, _NSPEC, _N)], axis=1)
    histp = jnp.transpose(hist, (0, 2, 1))
    histp = jnp.pad(histp, ((0, 0), (0, _NP - _N), (0, 0)))

    rel = rel_table[r0]
    hsh = hidden_states[h0]
    sth = score_text_embs[h0]
    b0r = b0.reshape(1, _D)
    b1r = b1.reshape(1, _D)
    blinr = b_lin.reshape(1, _D)
    bm1r = b_mlp1.reshape(1, 2 * _D)
    bm2r = b_mlp2.reshape(1, 1)

    nch = _NP // _CH
    full = lambda shape: pl.BlockSpec(shape, lambda b, i: (0,) * len(shape))
    aggsum, aggmax = pl.pallas_call(
        _dense_body,
        grid=(_B, nch),
        in_specs=[
            full((_NC, _NP, _NHIST)),
            full((_B, _D)), full((_B, _D)), full((_B, _D)),
            full((_D, _D)), full((2 * _D, _D)), full((_D, 2 * _D)),
            full((2 * _D, 1)), full((1, _D)), full((1, _D)),
            full((1, 2 * _D)), full((1, 1)),
            pl.BlockSpec(memory_space=pltpu.SMEM),
        ],
        out_specs=[
            pl.BlockSpec((1, _NEG, _D), lambda b, i: (b, 0, 0)),
            pl.BlockSpec((1, _NEG, _D), lambda b, i: (b, 0, 0)),
        ],
        out_shape=[
            jax.ShapeDtypeStruct((_B, _NEG, _D), jnp.float32),
            jax.ShapeDtypeStruct((_B, _NEG, _D), jnp.float32),
        ],
    )(histp, hsh, rel, sth, W0, W_lin, W_mlp1, W_mlp2,
      b0r, blinr, bm1r, bm2r, h0)

    fullf = lambda shape: pl.BlockSpec(shape, lambda: (0,) * len(shape))
    out = pl.pallas_call(
        _final_body,
        in_specs=[
            fullf((_B, _NEG, _D)), fullf((_B, _NEG, _D)),
            fullf((_NC, _NP, _NHIST)),
            fullf((_B, _D)), fullf((_B, _D)),
            fullf((_D, _D)), fullf((2 * _D, _D)), fullf((_D, 2 * _D)),
            fullf((2 * _D, 1)), fullf((1, _D)), fullf((1, _D)),
            fullf((1, 2 * _D)), fullf((1, 1)),
            pl.BlockSpec(memory_space=pltpu.SMEM),
            pl.BlockSpec(memory_space=pltpu.SMEM),
        ],
        out_specs=fullf((_B, _NEG)),
        out_shape=jax.ShapeDtypeStruct((_B, _NEG), jnp.float32),
    )(aggsum, aggmax, histp, hsh, rel, W1, W_lin, W_mlp1, W_mlp2,
      b1r, blinr, bm1r, bm2r, h0, t)
    return out


# speedup vs baseline: 317.9820x; 2.0138x over previous
import math

import jax
import jax.numpy as jnp
from jax import lax
from jax.experimental import pallas as pl
from jax.experimental.pallas import tpu as pltpu
from jax.experimental.pallas import tpu_sc as plsc

_N = 10000
_D = 128
_B = 2
_NEG = 4
_E = 160000
_CH = 1024
_NP = 10240
_NT = _B * _NEG
_NSPEC = _B + _NT
_NC = 2
_NSUB = 16
_NW = _NC * _NSUB
_EPW = _E // _NW
_DROWS = _NP // 16
_CSLEN = _NSPEC * _NP + 16


def _hist_body(ei_hbm, spec_hbm, zcs_hbm,
               out_deg_hbm, out_cs_hbm,
               e0_v, e1_v, spec_v, hist_v, ones_v, shared_cs):
    cid = lax.axis_index("c")
    sid = lax.axis_index("s")
    wid = sid * _NC + cid
    base = wid * _EPW

    @pl.when(sid == 0)
    def _():
        pltpu.sync_copy(zcs_hbm, shared_cs)

    pltpu.sync_copy(ei_hbm.at[pl.ds(base, _EPW)], e0_v)
    pltpu.sync_copy(ei_hbm.at[pl.ds(_E + base, _EPW)], e1_v)
    pltpu.sync_copy(spec_hbm, spec_v)
    pltpu.sync_copy(zcs_hbm.at[pl.ds(0, _NP)], hist_v)
    ones_v[...] = jnp.ones((16,), jnp.float32)
    plsc.subcore_barrier()

    ones = jnp.ones((16,), jnp.float32)
    lane = lax.iota(jnp.int32, 16)
    specs = [spec_v[s] for s in range(_NSPEC)]

    def body(j, carry):
        valid = lane < (_EPW - j * 16)
        a = e0_v[pl.ds(j * 16, 16)]
        b = e1_v[pl.ds(j * 16, 16)]
        plsc.addupdate_scatter(hist_v, [a], ones, mask=valid)
        plsc.addupdate_scatter(hist_v, [b], ones, mask=valid)

        hit = valid & (a == specs[0])
        for s in range(_NSPEC):
            hit = hit | (valid & (a == specs[s])) | (valid & (b == specs[s]))

        @pl.when(jnp.any(hit))
        def _():
            for s in range(_NSPEC):
                m0 = valid & (a == specs[s])

                @pl.when(jnp.any(m0))
                def _():
                    idx = jnp.where(m0, s * _NP + b, _NSPEC * _NP + lane)
                    pltpu.sync_copy(ones_v, shared_cs.at[idx], add=True)

                m1 = valid & (b == specs[s])

                @pl.when(jnp.any(m1))
                def _():
                    idx = jnp.where(m1, s * _NP + a, _NSPEC * _NP + lane)
                    pltpu.sync_copy(ones_v, shared_cs.at[idx], add=True)
        return carry

    lax.fori_loop(0, (_EPW + 15) // 16, body, 0)

    pltpu.sync_copy(hist_v, out_deg_hbm.at[wid])
    plsc.subcore_barrier()

    @pl.when(sid == 0)
    def _():
        pltpu.sync_copy(shared_cs, out_cs_hbm.at[cid])


def _sc_histograms(ei, spec):
    zcs = jnp.zeros((_CSLEN,), jnp.float32)
    mesh = plsc.VectorSubcoreMesh(core_axis_name="c", subcore_axis_name="s",
                                  num_cores=_NC, num_subcores=_NSUB)
    f = pl.kernel(
        _hist_body,
        out_type=(jax.ShapeDtypeStruct((_NW, _NP), jnp.float32),
                  jax.ShapeDtypeStruct((_NC, _CSLEN), jnp.float32)),
        mesh=mesh,
        compiler_params=pltpu.CompilerParams(needs_layout_passes=False),
        scratch_types=[
            pltpu.VMEM((_EPW,), jnp.int32),
            pltpu.VMEM((_EPW,), jnp.int32),
            pltpu.VMEM((_NSPEC, 16), jnp.int32),
            pltpu.VMEM((_NP,), jnp.float32),
            pltpu.VMEM((16,), jnp.float32),
            pltpu.VMEM_SHARED((_CSLEN,), jnp.float32),
        ],
    )
    return f(ei, spec, zcs)


_NCH = _NP // _CH


def _dense_body(degp_ref, csp_ref, hsht_ref, relt_ref, stht_ref,
                w0t_ref, wlht_ref, wlqt_ref, wm1t_ref, wm2t_ref,
                w1t_ref, b0t_ref, b1t_ref, blint_ref, bm1t_ref, bm2_ref,
                h0_ref, t_ref, out_ref, aggsum_ref, aggmax_ref, degt_ref,
                degsum_ref, mean_ref):
    b = pl.program_id(0)
    i = pl.program_id(1)

    @pl.when((b == 0) & (i == 0))
    def _():
        acc = degp_ref[0:1, :]
        for w in range(1, _NW):
            acc = acc + degp_ref[w:w + 1, :]
        degsum_ref[...] = acc
        mean_ref[0, 0] = jnp.sum(jnp.log(acc + 1.0)) / float(_N)

    mean_ld = mean_ref[0, 0]
    off = i * _CH
    deg = degsum_ref[0:1, pl.ds(off, _CH)]
    scal = jnp.log(deg + 1.0) / mean_ld
    degc = jnp.maximum(deg, 1.0)

    def csrow(s):
        return (csp_ref[0:1, pl.ds(s * _NP + off, _CH)]
                + csp_ref[1:2, pl.ds(s * _NP + off, _CH)])

    sel0 = jnp.where(b == 0, 1.0, 0.0)
    sel1 = jnp.where(b == 1, 1.0, 0.0)
    cb = sel0 * csrow(0) + sel1 * csrow(1)
    sb0 = jnp.concatenate([csrow(_B + k) for k in range(_NEG)], axis=0)
    sb1 = jnp.concatenate([csrow(_B + _NEG + k) for k in range(_NEG)], axis=0)
    sbt = sel0 * sb0 + sel1 * sb1

    relt = relt_ref[...]
    hsht = hsht_ref[...]
    stht = stht_ref[...]
    gate0 = jax.nn.sigmoid(jnp.sum(stht * relt, axis=0, keepdims=True)
                           / math.sqrt(float(_D)))
    mt = gate0 * hsht * relt
    ut = jnp.dot(w0t_ref[...], mt, preferred_element_type=jnp.float32)
    wt = jnp.dot(w0t_ref[...], jnp.maximum(mt, 0.0),
                 preferred_element_type=jnp.float32)
    qt = jnp.dot(wlqt_ref[...], relt, preferred_element_type=jnp.float32) \
        + blint_ref[...]
    bndt = hsht * relt

    u_b = sel0 * ut[:, 0:1] + sel1 * ut[:, 1:2]
    w_b = sel0 * wt[:, 0:1] + sel1 * wt[:, 1:2]
    q_b = sel0 * qt[:, 0:1] + sel1 * qt[:, 1:2]
    bnd_b = sel0 * bndt[:, 0:1] + sel1 * bndt[:, 1:2]
    h0_b = jnp.where(b == 0, h0_ref[0], h0_ref[1])

    a_coef = scal * cb / degc + jnp.where((cb > 0) & (cb == deg), scal, 0.0)
    b_coef = jnp.where((cb > 0) & (cb < deg), scal, 0.0)
    hid1 = jnp.maximum(u_b * a_coef + w_b * b_coef + b0t_ref[...], 0.0)
    lane_ids = lax.broadcasted_iota(jnp.int32, (1, _CH), 1) + off
    hid1 = hid1 + jnp.where(lane_ids == h0_b, 1.0, 0.0) * bnd_b

    z1 = jnp.maximum(jnp.dot(wlht_ref[...], hid1,
                             preferred_element_type=jnp.float32) + q_b, 0.0)
    z2 = jnp.maximum(jnp.dot(wm1t_ref[...], z1,
                             preferred_element_type=jnp.float32)
                     + bm1t_ref[...], 0.0)
    s1 = jnp.dot(wm2t_ref[...], z2, preferred_element_type=jnp.float32) \
        + bm2_ref[...]
    gate1 = jax.nn.sigmoid(s1)
    g = gate1 * hid1

    part_sum = lax.dot_general(g, sbt, (((1,), (1,)), ((), ())),
                               preferred_element_type=jnp.float32)
    neg_inf = jnp.float32(-jnp.inf)
    maxes = []
    degts = []
    for k in range(_NEG):
        gm = jnp.where(sbt[k:k + 1, :] > 0.0, g, neg_inf)
        maxes.append(jnp.max(gm, axis=1, keepdims=True))
        degts.append(jnp.sum(sbt[k:k + 1, :], axis=1, keepdims=True))
    part_max = jnp.concatenate(maxes, axis=1)
    part_degt = jnp.concatenate(degts, axis=1)

    for bb in range(_B):
        lo, hi = bb * _NEG, (bb + 1) * _NEG

        @pl.when((b == bb) & (i == 0))
        def _():
            aggsum_ref[:, lo:hi] = part_sum
            aggmax_ref[:, lo:hi] = part_max
            degt_ref[:, lo:hi] = part_degt

        @pl.when((b == bb) & (i > 0))
        def _():
            aggsum_ref[:, lo:hi] = aggsum_ref[:, lo:hi] + part_sum
            aggmax_ref[:, lo:hi] = jnp.maximum(aggmax_ref[:, lo:hi], part_max)
            degt_ref[:, lo:hi] = degt_ref[:, lo:hi] + part_degt

    @pl.when((b == _B - 1) & (i == _NCH - 1))
    def _():
        degt = degt_ref[...]
        scal_t = jnp.log(degt + 1.0) / mean_ld
        agg2 = (aggsum_ref[...] / jnp.maximum(degt, 1.0)
                + jnp.where(degt > 0, aggmax_ref[...], 0.0)) * scal_t
        hid2 = jnp.maximum(jnp.dot(w1t_ref[...], agg2,
                                   preferred_element_type=jnp.float32)
                           + b1t_ref[...], 0.0)

        bnd8 = jnp.concatenate(
            [bndt[:, bb:bb + 1] for bb in range(_B) for _ in range(_NEG)],
            axis=1)
        q8 = jnp.concatenate(
            [qt[:, bb:bb + 1] for bb in range(_B) for _ in range(_NEG)],
            axis=1)
        tmatch = jnp.concatenate(
            [jnp.where(t_ref[bb * _NEG + k] == h0_ref[bb],
                       1.0, 0.0).reshape(1, 1)
             for bb in range(_B) for k in range(_NEG)], axis=1)
        hid2 = hid2 + tmatch * bnd8

        z1f = jnp.maximum(jnp.dot(wlht_ref[...], hid2,
                                  preferred_element_type=jnp.float32)
                          + q8, 0.0)
        z2f = jnp.maximum(jnp.dot(wm1t_ref[...], z1f,
                                  preferred_element_type=jnp.float32)
                          + bm1t_ref[...], 0.0)
        s2 = jnp.dot(wm2t_ref[...], z2f,
                     preferred_element_type=jnp.float32) + bm2_ref[...]
        out_ref[...] = s2


def kernel(h_index, r_index, t_index, hidden_states, rel_hidden_states, x,
           edge_index, score_text_embs, all_index, rel_table, W0, b0, W1, b1,
           W_lin, b_lin, W_mlp1, b_mlp1, W_mlp2, b_mlp2):
    ei = edge_index.astype(jnp.int32).reshape(-1)
    h0 = h_index[:, 0].astype(jnp.int32)
    r0 = r_index[:, 0].astype(jnp.int32)
    t = t_index.astype(jnp.int32).reshape(-1)

    spec = jnp.tile(jnp.concatenate([h0, t])[:, None], (1, 16))
    degp, cs_part = _sc_histograms(ei, spec)

    relt = rel_table[r0].T
    hsht = hidden_states[h0].T
    stht = score_text_embs[h0].T

    full = lambda shape: pl.BlockSpec(shape, lambda b, i: (0,) * len(shape))
    out = pl.pallas_call(
        _dense_body,
        grid=(_B, _NCH),
        in_specs=[
            full((_NW, _NP)), full((_NC, _CSLEN)),
            full((_D, _B)), full((_D, _B)), full((_D, _B)),
            full((_D, _D)), full((_D, _D)), full((_D, _D)),
            full((2 * _D, _D)), full((1, 2 * _D)),
            full((_D, _D)), full((_D, 1)), full((_D, 1)), full((_D, 1)),
            full((2 * _D, 1)), full((1, 1)),
            pl.BlockSpec(memory_space=pltpu.SMEM),
            pl.BlockSpec(memory_space=pltpu.SMEM),
        ],
        out_specs=pl.BlockSpec((1, _NT), lambda b, i: (0, 0)),
        out_shape=jax.ShapeDtypeStruct((1, _NT), jnp.float32),
        scratch_shapes=[
            pltpu.VMEM((_D, _NT), jnp.float32),
            pltpu.VMEM((_D, _NT), jnp.float32),
            pltpu.VMEM((1, _NT), jnp.float32),
            pltpu.VMEM((1, _NP), jnp.float32),
            pltpu.SMEM((1, 1), jnp.float32),
        ],
    )(degp, cs_part, hsht, relt, stht,
      W0.T, W_lin[:_D].T, W_lin[_D:].T, W_mlp1.T, W_mlp2.T, W1.T,
      b0.reshape(_D, 1), b1.reshape(_D, 1), b_lin.reshape(_D, 1),
      b_mlp1.reshape(2 * _D, 1), b_mlp2.reshape(1, 1), h0, t)
    return out.reshape(_B, _NEG)


# speedup vs baseline: 381.1392x; 1.1986x over previous
import math

import jax
import jax.numpy as jnp
from jax import lax
from jax.experimental import pallas as pl
from jax.experimental.pallas import tpu as pltpu
from jax.experimental.pallas import tpu_sc as plsc

_N = 10000
_D = 128
_B = 2
_NEG = 4
_E = 160000
_CH = 2048
_NP = 10240
_NT = _B * _NEG
_NSPEC = _B + _NT
_NC = 2
_NSUB = 16
_NW = _NC * _NSUB
_EPW = _E // _NW
_DROWS = _NP // 16
_CSLEN = _NSPEC * _NP + 16


_UNROLL = 2


def _hist_body(ei_hbm, spec_hbm, zcs_hbm, zi_hbm,
               out_deg_hbm, out_cs_hbm,
               e0_v, e1_v, spec_v, hist_v, smark_v, ones_v, shared_cs):
    cid = lax.axis_index("c")
    sid = lax.axis_index("s")
    wid = sid * _NC + cid
    base = wid * _EPW

    @pl.when(sid == 0)
    def _():
        pltpu.sync_copy(zcs_hbm, shared_cs)

    pltpu.sync_copy(ei_hbm.at[pl.ds(base, _EPW)], e0_v.at[pl.ds(0, _EPW)])
    pltpu.sync_copy(ei_hbm.at[pl.ds(_E + base, _EPW)],
                    e1_v.at[pl.ds(0, _EPW)])
    pltpu.sync_copy(spec_hbm, spec_v)
    pltpu.sync_copy(zcs_hbm.at[pl.ds(0, _NP)], hist_v)
    pltpu.sync_copy(zi_hbm, smark_v)
    ones_v[...] = jnp.ones((16,), jnp.float32)

    ones = jnp.ones((16,), jnp.float32)
    lane = lax.iota(jnp.int32, 16)
    bitvals = jnp.where(lane < _NSPEC, jnp.left_shift(1, lane), 0)
    plsc.addupdate_scatter(smark_v, [spec_v[...]], bitvals)
    plsc.subcore_barrier()

    def halfbody(start):
        valid = lane < (_EPW - start)
        a = e0_v[pl.ds(start, 16)]
        b = e1_v[pl.ds(start, 16)]
        plsc.addupdate_scatter(hist_v, [a], ones, mask=valid)
        plsc.addupdate_scatter(hist_v, [b], ones, mask=valid)
        ma = plsc.load_gather(smark_v, [a], mask=valid)
        mb = plsc.load_gather(smark_v, [b], mask=valid)

        @pl.when(jnp.any(valid & ((ma | mb) != 0)))
        def _():
            for s in range(_NSPEC):
                bit = jnp.int32(1 << s)
                m0 = valid & ((ma & bit) != 0)

                @pl.when(jnp.any(m0))
                def _():
                    idx = jnp.where(m0, s * _NP + b, _NSPEC * _NP + lane)
                    pltpu.sync_copy(ones_v, shared_cs.at[idx], add=True)

                m1 = valid & ((mb & bit) != 0)

                @pl.when(jnp.any(m1))
                def _():
                    idx = jnp.where(m1, s * _NP + a, _NSPEC * _NP + lane)
                    pltpu.sync_copy(ones_v, shared_cs.at[idx], add=True)

    def body(j, carry):
        for k in range(_UNROLL):
            halfbody(j * 16 * _UNROLL + k * 16)
        return carry

    lax.fori_loop(0, (_EPW + 16 * _UNROLL - 1) // (16 * _UNROLL), body, 0)

    pltpu.sync_copy(hist_v, out_deg_hbm.at[wid])
    plsc.subcore_barrier()

    @pl.when(sid == 0)
    def _():
        pltpu.sync_copy(shared_cs, out_cs_hbm.at[cid])


def _sc_histograms(ei, spec):
    zcs = jnp.zeros((_CSLEN,), jnp.float32)
    zi = jnp.zeros((_NP,), jnp.int32)
    mesh = plsc.VectorSubcoreMesh(core_axis_name="c", subcore_axis_name="s",
                                  num_cores=_NC, num_subcores=_NSUB)
    epad = _EPW + 16 * _UNROLL
    f = pl.kernel(
        _hist_body,
        out_type=(jax.ShapeDtypeStruct((_NW, _NP), jnp.float32),
                  jax.ShapeDtypeStruct((_NC, _CSLEN), jnp.float32)),
        mesh=mesh,
        compiler_params=pltpu.CompilerParams(needs_layout_passes=False),
        scratch_types=[
            pltpu.VMEM((epad,), jnp.int32),
            pltpu.VMEM((epad,), jnp.int32),
            pltpu.VMEM((16,), jnp.int32),
            pltpu.VMEM((_NP,), jnp.float32),
            pltpu.VMEM((_NP,), jnp.int32),
            pltpu.VMEM((16,), jnp.float32),
            pltpu.VMEM_SHARED((_CSLEN,), jnp.float32),
        ],
    )
    return f(ei, spec, zcs, zi)


_NCH = _NP // _CH


def _c0dot(w_ref, x):
    return jnp.dot(w_ref[...], x, preferred_element_type=jnp.float32)


def _dense_body(degp_ref, csp_ref, hsht_ref, relt_ref, stht_ref,
                w0t_ref, wlht_ref, wlqt_ref, wm1t_ref, wm2t_ref, w1t_ref,
                b0t_ref, b1t_ref, blint_ref, bm1t_ref, bm2_ref,
                h0_ref, t_ref, out_ref, aggsum_ref, aggmax_ref, degt_ref,
                degsum_ref, const_ref, mean_ref):
    i = pl.program_id(0)

    @pl.when(i == 0)
    def _():
        acc = degp_ref[0:1, :]
        for w in range(1, _NW):
            acc = acc + degp_ref[w:w + 1, :]
        degsum_ref[...] = acc
        mean_ref[0, 0] = jnp.sum(jnp.log(acc + 1.0)) / float(_N)

        relt = relt_ref[...]
        hsht = hsht_ref[...]
        gate0 = jax.nn.sigmoid(
            jnp.sum(stht_ref[...] * relt, axis=0, keepdims=True)
            / math.sqrt(float(_D)))
        mt = gate0 * hsht * relt
        const_ref[:, 0:_B] = _c0dot(w0t_ref, mt)
        const_ref[:, _B:2 * _B] = _c0dot(w0t_ref, jnp.maximum(mt, 0.0))
        const_ref[:, 2 * _B:3 * _B] = \
            _c0dot(wlqt_ref, relt) + blint_ref[...]
        const_ref[:, 3 * _B:4 * _B] = hsht * relt

    mean_ld = mean_ref[0, 0]
    off = i * _CH
    deg = degsum_ref[0:1, pl.ds(off, _CH)]
    scal = jnp.log(deg + 1.0) / mean_ld
    degc = jnp.maximum(deg, 1.0)
    lane_ids = lax.broadcasted_iota(jnp.int32, (1, _CH), 1) + off

    def csrow(s):
        return (csp_ref[0:1, pl.ds(s * _NP + off, _CH)]
                + csp_ref[1:2, pl.ds(s * _NP + off, _CH)])

    for bb in range(_B):
        cb = csrow(bb)
        sbt = jnp.concatenate(
            [csrow(_B + bb * _NEG + k) for k in range(_NEG)], axis=0)
        u_b = const_ref[:, bb:bb + 1]
        w_b = const_ref[:, _B + bb:_B + bb + 1]
        q_b = const_ref[:, 2 * _B + bb:2 * _B + bb + 1]
        bnd_b = const_ref[:, 3 * _B + bb:3 * _B + bb + 1]

        a_coef = scal * cb / degc \
            + jnp.where((cb > 0) & (cb == deg), scal, 0.0)
        b_coef = jnp.where((cb > 0) & (cb < deg), scal, 0.0)
        hid1 = jnp.maximum(u_b * a_coef + w_b * b_coef + b0t_ref[...], 0.0)
        hid1 = hid1 + jnp.where(lane_ids == h0_ref[bb], 1.0, 0.0) * bnd_b

        z1 = jnp.maximum(_c0dot(wlht_ref, hid1) + q_b, 0.0)
        z2 = jnp.maximum(_c0dot(wm1t_ref, z1) + bm1t_ref[...], 0.0)
        s1 = _c0dot(wm2t_ref, z2) + bm2_ref[...]
        gate1 = jax.nn.sigmoid(s1)
        g = gate1 * hid1

        part_sum = lax.dot_general(g, sbt, (((1,), (1,)), ((), ())),
                                   preferred_element_type=jnp.float32)
        neg_inf = jnp.float32(-jnp.inf)
        maxes = []
        degts = []
        for k in range(_NEG):
            gm = jnp.where(sbt[k:k + 1, :] > 0.0, g, neg_inf)
            maxes.append(jnp.max(gm, axis=1, keepdims=True))
            degts.append(jnp.sum(sbt[k:k + 1, :], axis=1, keepdims=True))
        part_max = jnp.concatenate(maxes, axis=1)
        part_degt = jnp.concatenate(degts, axis=1)

        lo, hi = bb * _NEG, (bb + 1) * _NEG

        @pl.when(i == 0)
        def _():
            aggsum_ref[:, lo:hi] = part_sum
            aggmax_ref[:, lo:hi] = part_max
            degt_ref[:, lo:hi] = part_degt

        @pl.when(i > 0)
        def _():
            aggsum_ref[:, lo:hi] = aggsum_ref[:, lo:hi] + part_sum
            aggmax_ref[:, lo:hi] = jnp.maximum(aggmax_ref[:, lo:hi], part_max)
            degt_ref[:, lo:hi] = degt_ref[:, lo:hi] + part_degt

    @pl.when(i == _NCH - 1)
    def _():
        degt = degt_ref[...]
        scal_t = jnp.log(degt + 1.0) / mean_ld
        agg2 = (aggsum_ref[...] / jnp.maximum(degt, 1.0)
                + jnp.where(degt > 0, aggmax_ref[...], 0.0)) * scal_t
        hid2 = jnp.maximum(_c0dot(w1t_ref, agg2) + b1t_ref[...], 0.0)

        bnd8 = jnp.concatenate(
            [const_ref[:, 3 * _B + bb:3 * _B + bb + 1]
             for bb in range(_B) for _ in range(_NEG)], axis=1)
        q8 = jnp.concatenate(
            [const_ref[:, 2 * _B + bb:2 * _B + bb + 1]
             for bb in range(_B) for _ in range(_NEG)], axis=1)
        tmatch = jnp.concatenate(
            [jnp.where(t_ref[bb * _NEG + k] == h0_ref[bb],
                       1.0, 0.0).reshape(1, 1)
             for bb in range(_B) for k in range(_NEG)], axis=1)
        hid2 = hid2 + tmatch * bnd8

        z1f = jnp.maximum(_c0dot(wlht_ref, hid2) + q8, 0.0)
        z2f = jnp.maximum(_c0dot(wm1t_ref, z1f) + bm1t_ref[...], 0.0)
        s2 = _c0dot(wm2t_ref, z2f) + bm2_ref[...]
        out_ref[...] = s2


def kernel(h_index, r_index, t_index, hidden_states, rel_hidden_states, x,
           edge_index, score_text_embs, all_index, rel_table, W0, b0, W1, b1,
           W_lin, b_lin, W_mlp1, b_mlp1, W_mlp2, b_mlp2):
    ei = edge_index.astype(jnp.int32).reshape(-1)
    h0 = h_index[:, 0].astype(jnp.int32)
    r0 = r_index[:, 0].astype(jnp.int32)
    t = t_index.astype(jnp.int32).reshape(-1)

    spec = jnp.concatenate(
        [h0, t, _N + jnp.arange(16 - _NSPEC, dtype=jnp.int32)])
    degp, cs_part = _sc_histograms(ei, spec)

    relt = rel_table[r0].T
    hsht = hidden_states[h0].T
    stht = score_text_embs[h0].T

    full = lambda shape: pl.BlockSpec(shape, lambda i: (0,) * len(shape))
    out = pl.pallas_call(
        _dense_body,
        grid=(_NCH,),
        in_specs=[
            full((_NW, _NP)), full((_NC, _CSLEN)),
            full((_D, _B)), full((_D, _B)), full((_D, _B)),
            full((_D, _D)), full((_D, _D)), full((_D, _D)),
            full((2 * _D, _D)), full((1, 2 * _D)), full((_D, _D)),
            full((_D, 1)), full((_D, 1)), full((_D, 1)),
            full((2 * _D, 1)), full((1, 1)),
            pl.BlockSpec(memory_space=pltpu.SMEM),
            pl.BlockSpec(memory_space=pltpu.SMEM),
        ],
        out_specs=pl.BlockSpec((1, _NT), lambda i: (0, 0)),
        out_shape=jax.ShapeDtypeStruct((1, _NT), jnp.float32),
        scratch_shapes=[
            pltpu.VMEM((_D, _NT), jnp.float32),
            pltpu.VMEM((_D, _NT), jnp.float32),
            pltpu.VMEM((1, _NT), jnp.float32),
            pltpu.VMEM((1, _NP), jnp.float32),
            pltpu.VMEM((_D, 4 * _B), jnp.float32),
            pltpu.SMEM((1, 1), jnp.float32),
        ],
    )(degp, cs_part, hsht, relt, stht,
      W0.T, W_lin[:_D].T, W_lin[_D:].T, W_mlp1.T, W_mlp2.T, W1.T,
      b0.reshape(_D, 1), b1.reshape(_D, 1), b_lin.reshape(_D, 1),
      b_mlp1.reshape(2 * _D, 1), b_mlp2.reshape(1, 1), h0, t)
    return out.reshape(_B, _NEG)
